# initial kernel scaffold (unmeasured)
import jax
import jax.numpy as jnp
from jax import lax
from jax.experimental import pallas as pl
from jax.experimental.pallas import tpu as pltpu

N_DEV = 16


def _a2a(x_shard):
    m_tot, kb = x_shard.shape
    mb = m_tot // N_DEV

    def body(x_ref, out_ref, send_sems, recv_sems):
        me = lax.axis_index("i")
        out_ref[:, pl.ds(me * kb, kb)] = x_ref[pl.ds(me * mb, mb), :]
        rdmas = []
        for d in range(1, N_DEV):
            tgt = lax.rem(me + d, N_DEV)
            rdma = pltpu.make_async_remote_copy(
                src_ref=x_ref.at[pl.ds(tgt * mb, mb), :],
                dst_ref=out_ref.at[:, pl.ds(me * kb, kb)],
                send_sem=send_sems.at[d - 1],
                recv_sem=recv_sems.at[d - 1],
                device_id=(tgt,),
                device_id_type=pl.DeviceIdType.MESH,
            )
            rdma.start()
            rdmas.append(rdma)
        for r in rdmas:
            r.wait()

    return pl.pallas_call(
        body,
        out_shape=jax.ShapeDtypeStruct((mb, m_tot), x_shard.dtype),
        in_specs=[pl.BlockSpec(memory_space=pltpu.VMEM)],
        out_specs=pl.BlockSpec(memory_space=pltpu.VMEM),
        scratch_shapes=[
            pltpu.SemaphoreType.DMA((N_DEV - 1,)),
            pltpu.SemaphoreType.DMA((N_DEV - 1,)),
        ],
        compiler_params=pltpu.CompilerParams(collective_id=0),
    )(x_shard)


def _gemm_amax(x_full, w_mat):
    mb, k = x_full.shape
    _, n = w_mat.shape
    nt = 512
    steps = n // nt

    def body(x_ref, w_ref, y_ref, amax_ref):
        j = pl.program_id(0)
        y = jnp.dot(x_ref[...], w_ref[...], preferred_element_type=jnp.float32)
        y_ref[...] = y
        m = jnp.max(jnp.abs(y))

        @pl.when(j == 0)
        def _():
            amax_ref[0, 0] = m

        @pl.when(j != 0)
        def _():
            amax_ref[0, 0] = jnp.maximum(amax_ref[0, 0], m)

    return pl.pallas_call(
        body,
        grid=(steps,),
        in_specs=[
            pl.BlockSpec((mb, k), lambda j: (0, 0)),
            pl.BlockSpec((k, nt), lambda j: (0, j)),
        ],
        out_specs=[
            pl.BlockSpec((mb, nt), lambda j: (0, j)),
            pl.BlockSpec((1, 1), lambda j: (0, 0)),
        ],
        out_shape=[
            jax.ShapeDtypeStruct((mb, n), jnp.float32),
            jax.ShapeDtypeStruct((1, 1), jnp.float32),
        ],
    )(x_full, w_mat)


def _quant(y, amax):
    mb, n = y.shape

    def body(y_ref, amax_ref, out_ref, gbuf, send_sems, recv_sems):
        me = lax.axis_index("i")
        gbuf[pl.ds(me, 1), :] = amax_ref[...]
        rdmas = []
        for d in range(1, N_DEV):
            tgt = lax.rem(me + d, N_DEV)
            rdma = pltpu.make_async_remote_copy(
                src_ref=amax_ref,
                dst_ref=gbuf.at[pl.ds(me, 1), :],
                send_sem=send_sems.at[d - 1],
                recv_sem=recv_sems.at[d - 1],
                device_id=(tgt,),
                device_id_type=pl.DeviceIdType.MESH,
            )
            rdma.start()
            rdmas.append(rdma)
        for r in rdmas:
            r.wait()
        g = jnp.max(gbuf[...])
        scale = g / 448.0
        v = jnp.clip(y_ref[...] / scale, -448.0, 448.0)
        q = v.astype(jnp.float8_e4m3fn).astype(jnp.float32)
        out_ref[...] = q * scale

    return pl.pallas_call(
        body,
        out_shape=jax.ShapeDtypeStruct((mb, n), jnp.float32),
        in_specs=[
            pl.BlockSpec(memory_space=pltpu.VMEM),
            pl.BlockSpec(memory_space=pltpu.VMEM),
        ],
        out_specs=pl.BlockSpec(memory_space=pltpu.VMEM),
        scratch_shapes=[
            pltpu.VMEM((N_DEV, 1), jnp.float32),
            pltpu.SemaphoreType.DMA((N_DEV - 1,)),
            pltpu.SemaphoreType.DMA((N_DEV - 1,)),
        ],
        compiler_params=pltpu.CompilerParams(collective_id=1),
    )(y, amax)


def kernel(x, w_mat):
    x_full = _a2a(x)
    y, amax = _gemm_amax(x_full, w_mat)
    return _quant(y, amax)


# baseline (device time: 126748 ns/iter reference)
import jax
import jax.numpy as jnp
from jax import lax
from jax.experimental import pallas as pl
from jax.experimental.pallas import tpu as pltpu

N_DEV = 16


def _a2a(x_shard):
    m_tot, kb = x_shard.shape
    mb = m_tot // N_DEV

    def body(x_ref, out_ref, send_sems, recv_sems):
        me = lax.axis_index("i")
        out_ref[:, pl.ds(me * kb, kb)] = x_ref[pl.ds(me * mb, mb), :]
        rdmas = []
        for d in range(1, N_DEV):
            tgt = lax.rem(me + d, N_DEV)
            rdma = pltpu.make_async_remote_copy(
                src_ref=x_ref.at[pl.ds(tgt * mb, mb), :],
                dst_ref=out_ref.at[:, pl.ds(me * kb, kb)],
                send_sem=send_sems.at[d - 1],
                recv_sem=recv_sems.at[d - 1],
                device_id=(tgt,),
                device_id_type=pl.DeviceIdType.MESH,
            )
            rdma.start()
            rdmas.append(rdma)
        for r in rdmas:
            r.wait()

    return pl.pallas_call(
        body,
        out_shape=jax.ShapeDtypeStruct((mb, m_tot), x_shard.dtype),
        in_specs=[pl.BlockSpec(memory_space=pltpu.VMEM)],
        out_specs=pl.BlockSpec(memory_space=pltpu.VMEM),
        scratch_shapes=[
            pltpu.SemaphoreType.DMA((N_DEV - 1,)),
            pltpu.SemaphoreType.DMA((N_DEV - 1,)),
        ],
    )(x_shard)


def _gemm_amax(x_full, w_mat):
    mb, k = x_full.shape
    _, n = w_mat.shape
    nt = 512
    steps = n // nt

    def body(x_ref, w_ref, y_ref, amax_ref):
        j = pl.program_id(0)
        y = jnp.dot(x_ref[...], w_ref[...], preferred_element_type=jnp.float32)
        y_ref[...] = y
        m = jnp.max(jnp.abs(y))

        @pl.when(j == 0)
        def _():
            amax_ref[0, 0] = m

        @pl.when(j != 0)
        def _():
            amax_ref[0, 0] = jnp.maximum(amax_ref[0, 0], m)

    return pl.pallas_call(
        body,
        grid=(steps,),
        in_specs=[
            pl.BlockSpec((mb, k), lambda j: (0, 0)),
            pl.BlockSpec((k, nt), lambda j: (0, j)),
        ],
        out_specs=[
            pl.BlockSpec((mb, nt), lambda j: (0, j)),
            pl.BlockSpec((1, 1), lambda j: (0, 0), memory_space=pltpu.SMEM),
        ],
        out_shape=[
            jax.ShapeDtypeStruct((mb, n), jnp.float32),
            jax.ShapeDtypeStruct((1, 1), jnp.float32),
        ],
    )(x_full, w_mat)


def _quant(y, amax):
    mb, n = y.shape

    def body(y_ref, amax_ref, out_ref, gbuf, send_sems, recv_sems):
        me = lax.axis_index("i")
        gbuf[pl.ds(me, 1), :] = amax_ref[...]
        rdmas = []
        for d in range(1, N_DEV):
            tgt = lax.rem(me + d, N_DEV)
            rdma = pltpu.make_async_remote_copy(
                src_ref=amax_ref,
                dst_ref=gbuf.at[pl.ds(me, 1), :],
                send_sem=send_sems.at[d - 1],
                recv_sem=recv_sems.at[d - 1],
                device_id=(tgt,),
                device_id_type=pl.DeviceIdType.MESH,
            )
            rdma.start()
            rdmas.append(rdma)
        for r in rdmas:
            r.wait()
        g = jnp.max(gbuf[...])
        scale = g / 448.0
        v = jnp.clip(y_ref[...] / scale, -448.0, 448.0)
        q = v.astype(jnp.float8_e4m3fn).astype(jnp.float32)
        out_ref[...] = q * scale

    return pl.pallas_call(
        body,
        out_shape=jax.ShapeDtypeStruct((mb, n), jnp.float32),
        in_specs=[
            pl.BlockSpec(memory_space=pltpu.VMEM),
            pl.BlockSpec(memory_space=pltpu.VMEM),
        ],
        out_specs=pl.BlockSpec(memory_space=pltpu.VMEM),
        scratch_shapes=[
            pltpu.VMEM((N_DEV, 1), jnp.float32),
            pltpu.SemaphoreType.DMA((N_DEV - 1,)),
            pltpu.SemaphoreType.DMA((N_DEV - 1,)),
        ],
    )(y, amax)


def kernel(x, w_mat):
    x_full = _a2a(x)
    y, amax = _gemm_amax(x_full, w_mat)
    return _quant(y, amax)
